# baseline (speedup 1.0000x reference)
import functools

import jax
import jax.numpy as jnp
from jax import lax
from jax.experimental import pallas as pl
from jax.experimental.pallas import tpu as pltpu
from jax.experimental.pallas import tpu_sc as plsc

B = 16384
EMBED_K = 64
FEAT = 128

NC = 2
NS = 16
NW = NC * NS
B_PER_W = B // NW
CHUNK = 128
N_CHUNKS = B_PER_W // CHUNK


def _sc_gather_body(users_hbm, items_hbm, tu_hbm, f_hbm, theta_out, effe_out,
                    uidx_v, iidx_v, urows_v, irows_v, usem, isem):
    wid = lax.axis_index("s") * NC + lax.axis_index("c")
    base = wid * B_PER_W
    pltpu.sync_copy(users_hbm.at[pl.ds(base, B_PER_W)], uidx_v)
    pltpu.sync_copy(items_hbm.at[pl.ds(base, B_PER_W)], iidx_v)
    copies = []
    for c in range(N_CHUNKS):
        s = pl.ds(c * CHUNK, CHUNK)
        copies.append(pltpu.async_copy(tu_hbm.at[uidx_v.at[s]], urows_v.at[s], usem))
        copies.append(pltpu.async_copy(f_hbm.at[iidx_v.at[s]], irows_v.at[s], isem))
    for cp in copies:
        cp.wait()
    pltpu.sync_copy(urows_v, theta_out.at[pl.ds(base, B_PER_W)])
    pltpu.sync_copy(irows_v, effe_out.at[pl.ds(base, B_PER_W)])


@jax.jit
def _sc_gather(users, items, tu, f):
    mesh = plsc.VectorSubcoreMesh(core_axis_name="c", subcore_axis_name="s")
    return pl.kernel(
        _sc_gather_body,
        out_type=(
            jax.ShapeDtypeStruct((B, EMBED_K), jnp.float32),
            jax.ShapeDtypeStruct((B, FEAT), jnp.float32),
        ),
        mesh=mesh,
        compiler_params=pltpu.CompilerParams(use_tc_tiling_on_sc=False),
        scratch_types=[
            pltpu.VMEM((B_PER_W,), jnp.int32),
            pltpu.VMEM((B_PER_W,), jnp.int32),
            pltpu.VMEM((B_PER_W, EMBED_K), jnp.float32),
            pltpu.VMEM((B_PER_W, FEAT), jnp.float32),
            pltpu.SemaphoreType.DMA,
            pltpu.SemaphoreType.DMA,
        ],
    )(users, items, tu, f)


TC_BLK = 2048


def _tc_body(gu_ref, gi_ref, th_ref, ef_ref, w_ref, b_ref,
             xui_ref, proj_ref, gau_ref, gai_ref):
    e = ef_ref[...]
    mm = lax.dot_general(e, w_ref[...], (((1,), (1,)), ((), ())),
                         preferred_element_type=jnp.float32)
    p = mm + b_ref[...]
    n = jnp.sqrt(jnp.sum(p * p, axis=1, keepdims=True))
    p = p / jnp.maximum(n, 1e-12)
    proj_ref[...] = p
    gu = gu_ref[...]
    gi = gi_ref[...]
    gau_ref[...] = gu
    gai_ref[...] = gi
    xui = (jnp.sum(gu * gi, axis=1, keepdims=True)
           + jnp.sum(th_ref[...] * p, axis=1, keepdims=True))
    xui_ref[...] = xui


@jax.jit
def _tc_compute(gu, gi, theta_u, effe_i, w, b2d):
    grid = (B // TC_BLK,)
    row_blk = pl.BlockSpec((TC_BLK, EMBED_K), lambda i: (i, 0))
    return pl.pallas_call(
        _tc_body,
        grid=grid,
        in_specs=[
            row_blk,
            row_blk,
            row_blk,
            pl.BlockSpec((TC_BLK, FEAT), lambda i: (i, 0)),
            pl.BlockSpec((EMBED_K, FEAT), lambda i: (0, 0)),
            pl.BlockSpec((1, EMBED_K), lambda i: (0, 0)),
        ],
        out_specs=[
            pl.BlockSpec((TC_BLK, 1), lambda i: (i, 0)),
            row_blk,
            row_blk,
            row_blk,
        ],
        out_shape=[
            jax.ShapeDtypeStruct((B, 1), jnp.float32),
            jax.ShapeDtypeStruct((B, EMBED_K), jnp.float32),
            jax.ShapeDtypeStruct((B, EMBED_K), jnp.float32),
            jax.ShapeDtypeStruct((B, EMBED_K), jnp.float32),
        ],
    )(gu, gi, theta_u, effe_i, w, b2d)


def kernel(gu, gi, users, items, Tu, F, W, b):
    users32 = users.astype(jnp.int32)
    items32 = items.astype(jnp.int32)
    theta_u, effe_i = _sc_gather(users32, items32, Tu, F)
    xui2d, proj_i, gamma_u, gamma_i = _tc_compute(
        gu, gi, theta_u, effe_i, W, b.reshape(1, EMBED_K))
    xui = xui2d.reshape(B)
    return (xui, gamma_u, gamma_i, theta_u, proj_i)


# speedup vs baseline: 1.0935x; 1.0935x over previous
import functools

import jax
import jax.numpy as jnp
from jax import lax
from jax.experimental import pallas as pl
from jax.experimental.pallas import tpu as pltpu
from jax.experimental.pallas import tpu_sc as plsc

B = 16384
EMBED_K = 64
FEAT = 128

NC = 2
NS = 16
NW = NC * NS
B_PER_W = B // NW
CHUNK = 128
N_CHUNKS = B_PER_W // CHUNK


def _sc_gather_body(users_hbm, items_hbm, tu_hbm, f_hbm, theta_out, effe_out,
                    uidx_v, iidx_v, urows_v, irows_v, usem, isem):
    wid = lax.axis_index("s") * NC + lax.axis_index("c")
    base = wid * B_PER_W
    pltpu.sync_copy(users_hbm.at[pl.ds(base, B_PER_W)], uidx_v)
    pltpu.sync_copy(items_hbm.at[pl.ds(base, B_PER_W)], iidx_v)
    copies = []
    for c in range(N_CHUNKS):
        s = pl.ds(c * CHUNK, CHUNK)
        copies.append(pltpu.async_copy(tu_hbm.at[uidx_v.at[s]], urows_v.at[s], usem))
        copies.append(pltpu.async_copy(f_hbm.at[iidx_v.at[s]], irows_v.at[s], isem))
    for cp in copies:
        cp.wait()
    pltpu.sync_copy(urows_v, theta_out.at[pl.ds(base, B_PER_W)])
    pltpu.sync_copy(irows_v, effe_out.at[pl.ds(base, B_PER_W)])


@jax.jit
def _sc_gather(users, items, tu, f):
    mesh = plsc.VectorSubcoreMesh(core_axis_name="c", subcore_axis_name="s")
    return pl.kernel(
        _sc_gather_body,
        out_type=(
            jax.ShapeDtypeStruct((B, EMBED_K), jnp.float32),
            jax.ShapeDtypeStruct((B, FEAT), jnp.float32),
        ),
        mesh=mesh,
        compiler_params=pltpu.CompilerParams(use_tc_tiling_on_sc=False),
        scratch_types=[
            pltpu.VMEM((B_PER_W,), jnp.int32),
            pltpu.VMEM((B_PER_W,), jnp.int32),
            pltpu.VMEM((B_PER_W, EMBED_K), jnp.float32),
            pltpu.VMEM((B_PER_W, FEAT), jnp.float32),
            pltpu.SemaphoreType.DMA,
            pltpu.SemaphoreType.DMA,
        ],
    )(users, items, tu, f)


TC_BLK = 2048


def _tc_body(gu_ref, gi_ref, th_ref, ef_ref, w_ref, b_ref,
             xui_ref, proj_ref):
    e = ef_ref[...]
    mm = lax.dot_general(e, w_ref[...], (((1,), (1,)), ((), ())),
                         preferred_element_type=jnp.float32)
    p = mm + b_ref[...]
    n = jnp.sqrt(jnp.sum(p * p, axis=1, keepdims=True))
    p = p / jnp.maximum(n, 1e-12)
    proj_ref[...] = p
    xui = (jnp.sum(gu_ref[...] * gi_ref[...], axis=1, keepdims=True)
           + jnp.sum(th_ref[...] * p, axis=1, keepdims=True))
    xui_ref[...] = xui


@jax.jit
def _tc_compute(gu, gi, theta_u, effe_i, w, b2d):
    grid = (B // TC_BLK,)
    row_blk = pl.BlockSpec((TC_BLK, EMBED_K), lambda i: (i, 0))
    return pl.pallas_call(
        _tc_body,
        grid=grid,
        in_specs=[
            row_blk,
            row_blk,
            row_blk,
            pl.BlockSpec((TC_BLK, FEAT), lambda i: (i, 0)),
            pl.BlockSpec((EMBED_K, FEAT), lambda i: (0, 0)),
            pl.BlockSpec((1, EMBED_K), lambda i: (0, 0)),
        ],
        out_specs=[
            pl.BlockSpec((TC_BLK, 1), lambda i: (i, 0)),
            row_blk,
        ],
        out_shape=[
            jax.ShapeDtypeStruct((B, 1), jnp.float32),
            jax.ShapeDtypeStruct((B, EMBED_K), jnp.float32),
        ],
    )(gu, gi, theta_u, effe_i, w, b2d)


def kernel(gu, gi, users, items, Tu, F, W, b):
    users32 = users.astype(jnp.int32)
    items32 = items.astype(jnp.int32)
    theta_u, effe_i = _sc_gather(users32, items32, Tu, F)
    xui2d, proj_i = _tc_compute(
        gu, gi, theta_u, effe_i, W, b.reshape(1, EMBED_K))
    xui = xui2d.reshape(B)
    return (xui, gu, gi, theta_u, proj_i)


# speedup vs baseline: 1.4688x; 1.3432x over previous
import jax
import jax.numpy as jnp
from jax import lax
from jax.experimental import pallas as pl
from jax.experimental.pallas import tpu as pltpu
from jax.experimental.pallas import tpu_sc as plsc

B = 16384
EMBED_K = 64
FEAT = 128

NC = 2
NS = 16
NW = NC * NS
B_PER_W = B // NW
CHUNK = 128
N_CHUNKS = B_PER_W // CHUNK


def _sc_gather_body(users_hbm, items_hbm, tu_hbm, f_hbm, theta_out, effe_out,
                    uidx_v, iidx_v, urows_v, irows_v, usem, isem):
    wid = lax.axis_index("s") * NC + lax.axis_index("c")
    base = wid * B_PER_W
    pltpu.sync_copy(users_hbm.at[pl.ds(base, B_PER_W)], uidx_v)
    pltpu.sync_copy(items_hbm.at[pl.ds(base, B_PER_W)], iidx_v)
    copies = []
    for c in range(N_CHUNKS):
        s = pl.ds(c * CHUNK, CHUNK)
        copies.append(pltpu.async_copy(tu_hbm.at[uidx_v.at[s]], urows_v.at[s], usem))
        copies.append(pltpu.async_copy(f_hbm.at[iidx_v.at[s]], irows_v.at[s], isem))
    for cp in copies:
        cp.wait()
    pltpu.sync_copy(urows_v,
                    theta_out.at[pl.ds(base, B_PER_W), pl.ds(0, EMBED_K)])
    pltpu.sync_copy(irows_v, effe_out.at[pl.ds(base, B_PER_W)])


def _sc_gather(users, items, tu, f):
    mesh = plsc.VectorSubcoreMesh(core_axis_name="c", subcore_axis_name="s")
    return pl.kernel(
        _sc_gather_body,
        out_type=(
            jax.ShapeDtypeStruct((B, FEAT), jnp.float32),
            jax.ShapeDtypeStruct((B, FEAT), jnp.float32),
        ),
        mesh=mesh,
        compiler_params=pltpu.CompilerParams(use_tc_tiling_on_sc=False),
        scratch_types=[
            pltpu.VMEM((B_PER_W,), jnp.int32),
            pltpu.VMEM((B_PER_W,), jnp.int32),
            pltpu.VMEM((B_PER_W, EMBED_K), jnp.float32),
            pltpu.VMEM((B_PER_W, FEAT), jnp.float32),
            pltpu.SemaphoreType.DMA,
            pltpu.SemaphoreType.DMA,
        ],
    )(users, items, tu, f)


TC_BLK = 2048


def _tc_body(gut_ref, git_ref, th_ref, ef_ref, w_ref, b_ref,
             xui_ref, projt_ref, thetat_ref):
    e = ef_ref[...]
    mm = lax.dot_general(w_ref[...], e, (((1,), (1,)), ((), ())),
                         preferred_element_type=jnp.float32)
    p = mm + b_ref[...]
    n = jnp.sqrt(jnp.sum(p * p, axis=0, keepdims=True))
    p = p / jnp.maximum(n, 1e-12)
    projt_ref[...] = p
    th = th_ref[...][:, :EMBED_K]
    eye = jnp.eye(EMBED_K, dtype=jnp.float32)
    tht = lax.dot_general(eye, th, (((1,), (1,)), ((), ())),
                          preferred_element_type=jnp.float32)
    thetat_ref[...] = tht
    xui = (jnp.sum(gut_ref[...] * git_ref[...], axis=0, keepdims=True)
           + jnp.sum(tht * p, axis=0, keepdims=True))
    xui_ref[...] = xui


def _tc_compute(gut, git, theta128, effe_i, w, bcol):
    grid = (B // TC_BLK,)
    cm_blk = pl.BlockSpec((EMBED_K, TC_BLK), lambda i: (0, i))
    rm_blk = pl.BlockSpec((TC_BLK, FEAT), lambda i: (i, 0))
    return pl.pallas_call(
        _tc_body,
        grid=grid,
        in_specs=[
            cm_blk,
            cm_blk,
            rm_blk,
            rm_blk,
            pl.BlockSpec((EMBED_K, FEAT), lambda i: (0, 0)),
            pl.BlockSpec((EMBED_K, 1), lambda i: (0, 0)),
        ],
        out_specs=[
            pl.BlockSpec((1, TC_BLK), lambda i: (0, i)),
            cm_blk,
            cm_blk,
        ],
        out_shape=[
            jax.ShapeDtypeStruct((1, B), jnp.float32),
            jax.ShapeDtypeStruct((EMBED_K, B), jnp.float32),
            jax.ShapeDtypeStruct((EMBED_K, B), jnp.float32),
        ],
    )(gut, git, theta128, effe_i, w, bcol)


def kernel(gu, gi, users, items, Tu, F, W, b):
    users32 = users.astype(jnp.int32)
    items32 = items.astype(jnp.int32)
    theta128, effe_i = _sc_gather(users32, items32, Tu, F)
    xui2d, projt, thetat = _tc_compute(
        gu.T, gi.T, theta128, effe_i, W, b.reshape(EMBED_K, 1))
    xui = xui2d.reshape(B)
    return (xui, gu, gi, thetat.T, projt.T)


# speedup vs baseline: 1.5185x; 1.0338x over previous
import jax
import jax.numpy as jnp
from jax import lax
from jax.experimental import pallas as pl
from jax.experimental.pallas import tpu as pltpu
from jax.experimental.pallas import tpu_sc as plsc

B = 16384
EMBED_K = 64
FEAT = 128

NC = 2
NS = 16
NW = NC * NS
B_PER_W = B // NW
CHUNK = 128
N_CHUNKS = B_PER_W // CHUNK


def _sc_gather_body(users_hbm, items_hbm, tu_hbm, f_hbm, theta_out, effe_out,
                    uidx_v, iidx_v, urows_v, irows_v, usem, isem):
    wid = lax.axis_index("s") * NC + lax.axis_index("c")
    base = wid * B_PER_W
    pltpu.sync_copy(users_hbm.at[pl.ds(base, B_PER_W)], uidx_v)
    pltpu.sync_copy(items_hbm.at[pl.ds(base, B_PER_W)], iidx_v)
    copies = []
    for c in range(N_CHUNKS):
        s = pl.ds(c * CHUNK, CHUNK)
        copies.append(pltpu.async_copy(tu_hbm.at[uidx_v.at[s]], urows_v.at[s], usem))
        copies.append(pltpu.async_copy(f_hbm.at[iidx_v.at[s]], irows_v.at[s], isem))
    for cp in copies:
        cp.wait()
    pltpu.sync_copy(urows_v,
                    theta_out.at[pl.ds(base, B_PER_W), pl.ds(0, EMBED_K)])
    pltpu.sync_copy(irows_v, effe_out.at[pl.ds(base, B_PER_W)])


def _sc_gather(users, items, tu, f):
    mesh = plsc.VectorSubcoreMesh(core_axis_name="c", subcore_axis_name="s")
    return pl.kernel(
        _sc_gather_body,
        out_type=(
            jax.ShapeDtypeStruct((B, FEAT), jnp.float32),
            jax.ShapeDtypeStruct((B, FEAT), jnp.float32),
        ),
        mesh=mesh,
        compiler_params=pltpu.CompilerParams(use_tc_tiling_on_sc=False),
        scratch_types=[
            pltpu.VMEM((B_PER_W,), jnp.int32),
            pltpu.VMEM((B_PER_W,), jnp.int32),
            pltpu.VMEM((B_PER_W, EMBED_K), jnp.float32),
            pltpu.VMEM((B_PER_W, FEAT), jnp.float32),
            pltpu.SemaphoreType.DMA,
            pltpu.SemaphoreType.DMA,
        ],
    )(users, items, tu, f)


TC_BLK = 2048


def _tc_body(gut_ref, git_ref, th_ref, ef_ref, w_ref, b_ref,
             xui_ref, projt_ref, thetat_ref, gaut_ref, gait_ref):
    e = ef_ref[...]
    mm = lax.dot_general(w_ref[...], e, (((1,), (1,)), ((), ())),
                         preferred_element_type=jnp.float32)
    p = mm + b_ref[...]
    n = jnp.sqrt(jnp.sum(p * p, axis=0, keepdims=True))
    p = p / jnp.maximum(n, 1e-12)
    projt_ref[...] = p
    th = th_ref[...][:, :EMBED_K]
    eye = jnp.eye(EMBED_K, dtype=jnp.float32)
    tht = lax.dot_general(eye, th, (((1,), (1,)), ((), ())),
                          preferred_element_type=jnp.float32)
    thetat_ref[...] = tht
    gut = gut_ref[...]
    git = git_ref[...]
    gaut_ref[...] = gut
    gait_ref[...] = git
    xui = (jnp.sum(gut * git, axis=0, keepdims=True)
           + jnp.sum(tht * p, axis=0, keepdims=True))
    xui_ref[...] = xui


def _tc_compute(gut, git, theta128, effe_i, w, bcol):
    grid = (B // TC_BLK,)
    cm_blk = pl.BlockSpec((EMBED_K, TC_BLK), lambda i: (0, i))
    rm_blk = pl.BlockSpec((TC_BLK, FEAT), lambda i: (i, 0))
    return pl.pallas_call(
        _tc_body,
        grid=grid,
        in_specs=[
            cm_blk,
            cm_blk,
            rm_blk,
            rm_blk,
            pl.BlockSpec((EMBED_K, FEAT), lambda i: (0, 0)),
            pl.BlockSpec((EMBED_K, 1), lambda i: (0, 0)),
        ],
        out_specs=[
            pl.BlockSpec((1, TC_BLK), lambda i: (0, i)),
            cm_blk,
            cm_blk,
            cm_blk,
            cm_blk,
        ],
        out_shape=[
            jax.ShapeDtypeStruct((1, B), jnp.float32),
            jax.ShapeDtypeStruct((EMBED_K, B), jnp.float32),
            jax.ShapeDtypeStruct((EMBED_K, B), jnp.float32),
            jax.ShapeDtypeStruct((EMBED_K, B), jnp.float32),
            jax.ShapeDtypeStruct((EMBED_K, B), jnp.float32),
        ],
    )(gut, git, theta128, effe_i, w, bcol)


def kernel(gu, gi, users, items, Tu, F, W, b):
    users32 = users.astype(jnp.int32)
    items32 = items.astype(jnp.int32)
    theta128, effe_i = _sc_gather(users32, items32, Tu, F)
    xui2d, projt, thetat, gaut, gait = _tc_compute(
        gu.T, gi.T, theta128, effe_i, W, b.reshape(EMBED_K, 1))
    xui = xui2d.reshape(B)
    return (xui, gaut.T, gait.T, thetat.T, projt.T)


# speedup vs baseline: 1.5729x; 1.0358x over previous
import jax
import jax.numpy as jnp
from jax import lax
from jax.experimental import pallas as pl
from jax.experimental.pallas import tpu as pltpu
from jax.experimental.pallas import tpu_sc as plsc

B = 16384
EMBED_K = 64
FEAT = 128

NC = 2
NS = 16
NW = NC * NS
B_PER_W = B // NW
CHUNK = 128
N_CHUNKS = B_PER_W // CHUNK


def _sc_gather_f_body(items_hbm, f_hbm, effe_out, iidx_v, irows_v, isem):
    wid = lax.axis_index("s") * NC + lax.axis_index("c")
    base = wid * B_PER_W
    pltpu.sync_copy(items_hbm.at[pl.ds(base, B_PER_W)], iidx_v)
    copies = []
    for c in range(N_CHUNKS):
        s = pl.ds(c * CHUNK, CHUNK)
        copies.append(pltpu.async_copy(f_hbm.at[iidx_v.at[s]], irows_v.at[s], isem))
    for cp in copies:
        cp.wait()
    pltpu.sync_copy(irows_v, effe_out.at[pl.ds(base, B_PER_W)])


def _sc_gather_f(items, f):
    mesh = plsc.VectorSubcoreMesh(core_axis_name="c", subcore_axis_name="s")
    return pl.kernel(
        _sc_gather_f_body,
        out_type=jax.ShapeDtypeStruct((B, FEAT), jnp.float32),
        mesh=mesh,
        compiler_params=pltpu.CompilerParams(use_tc_tiling_on_sc=False),
        scratch_types=[
            pltpu.VMEM((B_PER_W,), jnp.int32),
            pltpu.VMEM((B_PER_W, FEAT), jnp.float32),
            pltpu.SemaphoreType.DMA,
        ],
    )(items, f)


def _sc_gather_tu_body(users_hbm, tu_hbm, theta_out, uidx_v, urows_v, usem):
    wid = lax.axis_index("s") * NC + lax.axis_index("c")
    base = wid * B_PER_W
    pltpu.sync_copy(users_hbm.at[pl.ds(base, B_PER_W)], uidx_v)
    copies = []
    for c in range(N_CHUNKS):
        s = pl.ds(c * CHUNK, CHUNK)
        copies.append(pltpu.async_copy(tu_hbm.at[uidx_v.at[s]], urows_v.at[s], usem))
    for cp in copies:
        cp.wait()
    pltpu.sync_copy(urows_v,
                    theta_out.at[pl.ds(base, B_PER_W), pl.ds(0, EMBED_K)])


def _sc_gather_tu(users, tu):
    mesh = plsc.VectorSubcoreMesh(core_axis_name="c", subcore_axis_name="s")
    return pl.kernel(
        _sc_gather_tu_body,
        out_type=jax.ShapeDtypeStruct((B, FEAT), jnp.float32),
        mesh=mesh,
        compiler_params=pltpu.CompilerParams(use_tc_tiling_on_sc=False),
        scratch_types=[
            pltpu.VMEM((B_PER_W,), jnp.int32),
            pltpu.VMEM((B_PER_W, EMBED_K), jnp.float32),
            pltpu.SemaphoreType.DMA,
        ],
    )(users, tu)


TC_BLK = 2048


def _tc_body(gut_ref, git_ref, th_ref, ef_ref, w_ref, b_ref,
             xui_ref, projt_ref, thetat_ref, gaut_ref, gait_ref):
    e = ef_ref[...]
    mm = lax.dot_general(w_ref[...], e, (((1,), (1,)), ((), ())),
                         preferred_element_type=jnp.float32)
    p = mm + b_ref[...]
    n = jnp.sqrt(jnp.sum(p * p, axis=0, keepdims=True))
    p = p / jnp.maximum(n, 1e-12)
    projt_ref[...] = p
    th = th_ref[...][:, :EMBED_K]
    eye = jnp.eye(EMBED_K, dtype=jnp.float32)
    tht = lax.dot_general(eye, th, (((1,), (1,)), ((), ())),
                          preferred_element_type=jnp.float32)
    thetat_ref[...] = tht
    gut = gut_ref[...]
    git = git_ref[...]
    gaut_ref[...] = gut
    gait_ref[...] = git
    xui = (jnp.sum(gut * git, axis=0, keepdims=True)
           + jnp.sum(tht * p, axis=0, keepdims=True))
    xui_ref[...] = xui


def _tc_compute(gut, git, theta128, effe_i, w, bcol):
    grid = (B // TC_BLK,)
    cm_blk = pl.BlockSpec((EMBED_K, TC_BLK), lambda i: (0, i))
    rm_blk = pl.BlockSpec((TC_BLK, FEAT), lambda i: (i, 0))
    return pl.pallas_call(
        _tc_body,
        grid=grid,
        in_specs=[
            cm_blk,
            cm_blk,
            rm_blk,
            rm_blk,
            pl.BlockSpec((EMBED_K, FEAT), lambda i: (0, 0)),
            pl.BlockSpec((EMBED_K, 1), lambda i: (0, 0)),
        ],
        out_specs=[
            pl.BlockSpec((1, TC_BLK), lambda i: (0, i)),
            cm_blk,
            cm_blk,
            cm_blk,
            cm_blk,
        ],
        out_shape=[
            jax.ShapeDtypeStruct((1, B), jnp.float32),
            jax.ShapeDtypeStruct((EMBED_K, B), jnp.float32),
            jax.ShapeDtypeStruct((EMBED_K, B), jnp.float32),
            jax.ShapeDtypeStruct((EMBED_K, B), jnp.float32),
            jax.ShapeDtypeStruct((EMBED_K, B), jnp.float32),
        ],
    )(gut, git, theta128, effe_i, w, bcol)


def kernel(gu, gi, users, items, Tu, F, W, b):
    users32 = users.astype(jnp.int32)
    items32 = items.astype(jnp.int32)
    effe_i = _sc_gather_f(items32, F)
    theta128 = _sc_gather_tu(users32, Tu)
    xui2d, projt, thetat, gaut, gait = _tc_compute(
        gu.T, gi.T, theta128, effe_i, W, b.reshape(EMBED_K, 1))
    xui = xui2d.reshape(B)
    return (xui, gaut.T, gait.T, thetat.T, projt.T)
